# Initial kernel scaffold; baseline (speedup 1.0000x reference)
import functools

import jax
import jax.numpy as jnp
import numpy as np
from jax import lax
from jax.experimental import pallas as pl
from jax.experimental.pallas import tpu as pltpu
from jax.experimental.pallas import tpu_sc as plsc

N_IMG = 256
NPLANES = 64
NK = 262144
CHUNK = 16384
UNROLL = 8


def _dft_mats():
    k = np.arange(N_IMG) - N_IMG // 2
    m = np.outer(k, k).astype(np.float64)
    f = np.exp(-2j * np.pi * m / N_IMG) / np.sqrt(N_IMG)
    return (jnp.asarray(f.real, dtype=jnp.float32),
            jnp.asarray(f.imag, dtype=jnp.float32))


def _fft_body(fr_ref, fi_ref, x_ref, out_ref):
    x = x_ref[0]
    fr = fr_ref[...]
    fi = fi_ref[...]
    ar = jnp.dot(fr, x, preferred_element_type=jnp.float32)
    ai = jnp.dot(fi, x, preferred_element_type=jnp.float32)
    out_ref[0, 0] = (jnp.dot(ar, fr, preferred_element_type=jnp.float32)
                     - jnp.dot(ai, fi, preferred_element_type=jnp.float32))
    out_ref[1, 0] = (jnp.dot(ar, fi, preferred_element_type=jnp.float32)
                     + jnp.dot(ai, fr, preferred_element_type=jnp.float32))


def _centered_fft2(img_flat):
    fr, fi = _dft_mats()
    nb = img_flat.shape[0]
    return pl.pallas_call(
        _fft_body,
        grid=(nb,),
        in_specs=[
            pl.BlockSpec((N_IMG, N_IMG), lambda b: (0, 0)),
            pl.BlockSpec((N_IMG, N_IMG), lambda b: (0, 0)),
            pl.BlockSpec((1, N_IMG, N_IMG), lambda b: (b, 0, 0)),
        ],
        out_specs=pl.BlockSpec((2, 1, N_IMG, N_IMG), lambda b: (0, b, 0, 0)),
        out_shape=jax.ShapeDtypeStruct((2, nb, N_IMG, N_IMG), jnp.float32),
    )(fr, fi, img_flat)


def _sc_gather(grids, idx):
    mesh = plsc.VectorSubcoreMesh(core_axis_name="c", subcore_axis_name="s")

    @functools.partial(
        pl.kernel,
        mesh=mesh,
        out_type=jax.ShapeDtypeStruct((NPLANES, NK), jnp.float32),
        scratch_types=[
            pltpu.VMEM((N_IMG * N_IMG,), jnp.float32),
            pltpu.VMEM((CHUNK,), jnp.int32),
            pltpu.VMEM((CHUNK,), jnp.float32),
        ],
    )
    def gather_kernel(grids_hbm, idx_hbm, out_hbm, grid_v, idx_v, out_v):
        wid = lax.axis_index("s") * 2 + lax.axis_index("c")
        for pp in range(NPLANES // 32):
            p = wid * 2 + pp
            n = (p % 32) // 8
            pltpu.sync_copy(grids_hbm.at[p], grid_v)

            def chunk_body(kc, _, n=n, p=p):
                base = kc * CHUNK
                pltpu.sync_copy(idx_hbm.at[n, pl.ds(base, CHUNK)], idx_v)

                def g_body(j, _):
                    b0 = j * (16 * UNROLL)
                    for u in range(UNROLL):
                        off = b0 + u * 16
                        iv = idx_v[pl.ds(off, 16)]
                        out_v[pl.ds(off, 16)] = plsc.load_gather(grid_v, [iv])
                    return 0

                lax.fori_loop(0, CHUNK // (16 * UNROLL), g_body, 0)
                pltpu.sync_copy(out_v, out_hbm.at[p, pl.ds(base, CHUNK)])
                return 0

            lax.fori_loop(0, NK // CHUNK, chunk_body, 0)

    return gather_kernel(grids, idx)


def kernel(img, trj):
    nb, nc = img.shape[0], img.shape[1]
    img_flat = img.reshape(nb * nc, N_IMG, N_IMG)
    grids = _centered_fft2(img_flat)
    grids = grids.reshape(NPLANES, N_IMG * N_IMG)
    idx = trj[..., 0] * N_IMG + trj[..., 1]
    out = _sc_gather(grids, idx)
    out = out.reshape(2, nb, nc, NK)
    return lax.complex(out[0], out[1])


# speedup vs baseline: 43.4107x; 43.4107x over previous
import functools

import jax
import jax.numpy as jnp
import numpy as np
from jax import lax
from jax.experimental import pallas as pl
from jax.experimental.pallas import tpu as pltpu
from jax.experimental.pallas import tpu_sc as plsc

N_IMG = 256
NPLANES = 64
NK = 262144
CHUNK = 16384
UNROLL = 8


def _dft_mats():
    k = np.arange(N_IMG) - N_IMG // 2
    m = np.outer(k, k).astype(np.float64)
    f = np.exp(-2j * np.pi * m / N_IMG) / np.sqrt(N_IMG)
    return (jnp.asarray(f.real, dtype=jnp.float32),
            jnp.asarray(f.imag, dtype=jnp.float32))


def _fft_body(fr_ref, fi_ref, x_ref, out_ref):
    x = x_ref[0]
    fr = fr_ref[...]
    fi = fi_ref[...]
    ar = jnp.dot(fr, x, preferred_element_type=jnp.float32)
    ai = jnp.dot(fi, x, preferred_element_type=jnp.float32)
    out_ref[0, 0] = (jnp.dot(ar, fr, preferred_element_type=jnp.float32)
                     - jnp.dot(ai, fi, preferred_element_type=jnp.float32))
    out_ref[1, 0] = (jnp.dot(ar, fi, preferred_element_type=jnp.float32)
                     + jnp.dot(ai, fr, preferred_element_type=jnp.float32))


def _centered_fft2(img_flat):
    fr, fi = _dft_mats()
    nb = img_flat.shape[0]
    return pl.pallas_call(
        _fft_body,
        grid=(nb,),
        in_specs=[
            pl.BlockSpec((N_IMG, N_IMG), lambda b: (0, 0)),
            pl.BlockSpec((N_IMG, N_IMG), lambda b: (0, 0)),
            pl.BlockSpec((1, N_IMG, N_IMG), lambda b: (b, 0, 0)),
        ],
        out_specs=pl.BlockSpec((2, 1, N_IMG, N_IMG), lambda b: (0, b, 0, 0)),
        out_shape=jax.ShapeDtypeStruct((2, nb, N_IMG, N_IMG), jnp.float32),
    )(fr, fi, img_flat)


def _sc_gather(grids, idx):
    mesh = plsc.VectorSubcoreMesh(core_axis_name="c", subcore_axis_name="s")

    @functools.partial(
        pl.kernel,
        mesh=mesh,
        out_type=jax.ShapeDtypeStruct((NPLANES, NK), jnp.float32),
        scratch_types=[
            pltpu.VMEM((N_IMG * N_IMG,), jnp.float32),
            pltpu.VMEM((CHUNK,), jnp.int32),
            pltpu.VMEM((CHUNK,), jnp.float32),
        ],
        compiler_params=pltpu.CompilerParams(needs_layout_passes=False),
    )
    def gather_kernel(grids_hbm, idx_hbm, out_hbm, grid_v, idx_v, out_v):
        wid = lax.axis_index("s") * 2 + lax.axis_index("c")
        for pp in range(NPLANES // 32):
            p = wid * 2 + pp
            n = (p % 32) // 8
            pltpu.sync_copy(grids_hbm.at[p], grid_v)

            def chunk_body(kc, _, n=n, p=p):
                base = kc * CHUNK
                pltpu.sync_copy(idx_hbm.at[n, pl.ds(base, CHUNK)], idx_v)

                def g_body(j, _):
                    b0 = j * (16 * UNROLL)
                    for u in range(UNROLL):
                        off = b0 + u * 16
                        iv = idx_v[pl.ds(off, 16)]
                        out_v[pl.ds(off, 16)] = plsc.load_gather(grid_v, [iv])
                    return 0

                lax.fori_loop(0, CHUNK // (16 * UNROLL), g_body, 0)
                pltpu.sync_copy(out_v, out_hbm.at[p, pl.ds(base, CHUNK)])
                return 0

            lax.fori_loop(0, NK // CHUNK, chunk_body, 0)

    return gather_kernel(grids, idx)


def kernel(img, trj):
    nb, nc = img.shape[0], img.shape[1]
    img_flat = img.reshape(nb * nc, N_IMG, N_IMG)
    grids = _centered_fft2(img_flat)
    grids = grids.reshape(NPLANES, N_IMG * N_IMG)
    idx = trj[..., 0] * N_IMG + trj[..., 1]
    out = _sc_gather(grids, idx)
    out = out.reshape(2, nb, nc, NK)
    return lax.complex(out[0], out[1])
